# baseline (speedup 1.0000x reference)
import functools

import jax
import jax.numpy as jnp
from jax import lax
from jax.experimental import pallas as pl
from jax.experimental.pallas import tpu as pltpu
from jax.experimental.pallas import tpu_sc as plsc

B, A, C, M = 4, 50000, 16, 64
NW = 32
CHUNK = 1568
NSTRIP = CHUNK // 16
LAST_START = A - CHUNK


@functools.partial(
    pl.kernel,
    out_type=(jax.ShapeDtypeStruct((B * A,), jnp.float32),
              jax.ShapeDtypeStruct((NW * 16,), jnp.float32)),
    mesh=plsc.VectorSubcoreMesh(core_axis_name="c", subcore_axis_name="s"),
    scratch_types=[
        pltpu.VMEM((CHUNK * 3,), jnp.float32),
        pltpu.VMEM((CHUNK * 3,), jnp.float32),
        pltpu.VMEM((4 * M,), jnp.float32),
        pltpu.VMEM((CHUNK,), jnp.float32),
        pltpu.VMEM((16,), jnp.float32),
    ],
    compiler_params=pltpu.CompilerParams(needs_layout_passes=False),
)
def _match_sc(reg_hbm, anc_hbm, ann_hbm, code_hbm, out_hbm,
              anc_v, reg_v, ann_v, code_v, res_v):
    wid = lax.axis_index("s") * 2 + lax.axis_index("c")
    start = jnp.minimum(wid * CHUNK, LAST_START)
    own_lo = wid * CHUNK

    iota = lax.iota(jnp.int32, 16)
    zeros_i = iota * 0

    pltpu.sync_copy(anc_hbm.at[pl.ds(start * 3, CHUNK * 3)], anc_v)

    def strip_tail(base, aidx, d2min, bidx4, acc):
        npos_acc, xy_acc, ang_acc = acc
        aal = plsc.load_gather(anc_v, [aidx + 2])
        bx = plsc.load_gather(ann_v, [bidx4])
        by = plsc.load_gather(ann_v, [bidx4 + 1])
        bal = plsc.load_gather(ann_v, [bidx4 + 2])
        blb = plsc.load_gather(ann_v, [bidx4 + 3])
        aa = jnp.abs(aal - bal)

        pos_r = (d2min <= 25.0) & (aa <= 10.0)
        t0 = (d2min >= 56.25) | (aa >= 15.0)
        code = jnp.where(pos_r, blb, jnp.where(t0, 16.0, -1.0))
        code_v[pl.ds(base, 16)] = code

        validm = (start + base + iota) >= own_lo
        pos = pos_r & validm
        npos_acc = npos_acc + jnp.where(pos, 1.0, 0.0)

        ax = plsc.load_gather(anc_v, [aidx])
        ay = plsc.load_gather(anc_v, [aidx + 1])
        r0 = plsc.load_gather(reg_v, [aidx])
        r1 = plsc.load_gather(reg_v, [aidx + 1])
        r2 = plsc.load_gather(reg_v, [aidx + 2])
        dxr = jnp.abs((bx - ax) - r0)
        dyr = jnp.abs((by - ay) - r1)
        lx = jnp.where(dxr <= 1.0 / 9.0, 4.5 * dxr * dxr, dxr - 0.5 / 9.0)
        ly = jnp.where(dyr <= 1.0 / 9.0, 4.5 * dyr * dyr, dyr - 0.5 / 9.0)
        da = (jnp.abs((bal - aal) - r2) - 10.0) / 5.0
        da = jnp.where(da <= 0.0, 0.0, da)
        posf = jnp.where(pos, 1.0, 0.0)
        xy_acc = xy_acc + (lx + ly) * posf
        ang_acc = ang_acc + da * posf
        return npos_acc, xy_acc, ang_acc

    def batch_body(j, resvec):
        pltpu.sync_copy(reg_hbm.at[pl.ds(j * (3 * A) + start * 3, CHUNK * 3)], reg_v)
        pltpu.sync_copy(ann_hbm.at[pl.ds(j * (4 * M), 4 * M)], ann_v)

        def group_body(g, acc):
            base0 = g * 32
            base1 = base0 + 16
            aidx0 = iota * 3 + base0 * 3
            aidx1 = aidx0 + 48
            ax0 = plsc.load_gather(anc_v, [aidx0])
            ay0 = plsc.load_gather(anc_v, [aidx0 + 1])
            ax1 = plsc.load_gather(anc_v, [aidx1])
            ay1 = plsc.load_gather(anc_v, [aidx1 + 1])

            def m_body(m, mc):
                d0, b0, d1, b1 = mc
                mv = zeros_i + m * 4
                gx = plsc.load_gather(ann_v, [mv])
                gy = plsc.load_gather(ann_v, [mv + 1])
                dx0 = ax0 - gx
                dy0 = ay0 - gy
                dd0 = dx0 * dx0 + dy0 * dy0
                dx1 = ax1 - gx
                dy1 = ay1 - gy
                dd1 = dx1 * dx1 + dy1 * dy1
                lt0 = dd0 < d0
                lt1 = dd1 < d1
                return (jnp.where(lt0, dd0, d0), jnp.where(lt0, mv, b0),
                        jnp.where(lt1, dd1, d1), jnp.where(lt1, mv, b1))

            inf = jnp.full((16,), jnp.inf, jnp.float32)
            d0, b0, d1, b1 = lax.fori_loop(0, M, m_body,
                                           (inf, zeros_i, inf, zeros_i))
            acc = strip_tail(base0, aidx0, d0, b0, acc)
            acc = strip_tail(base1, aidx1, d1, b1, acc)
            return acc

        zf = jnp.zeros((16,), jnp.float32)
        npos_acc, xy_acc, ang_acc = lax.fori_loop(
            0, NSTRIP // 2, group_body, (zf, zf, zf))

        pltpu.sync_copy(code_v, code_hbm.at[pl.ds(j * A + start, CHUNK)])

        resvec = jnp.where(iota == 4 * j + 1, jnp.sum(npos_acc), resvec)
        resvec = jnp.where(iota == 4 * j + 2, jnp.sum(xy_acc), resvec)
        resvec = jnp.where(iota == 4 * j + 3, jnp.sum(ang_acc), resvec)
        return resvec

    res_v[...] = lax.fori_loop(0, B, batch_body, jnp.zeros((16,), jnp.float32))
    pltpu.sync_copy(res_v, out_hbm.at[pl.ds(wid * 16, 16)])


BA_BLK = 2000
NBLK = A // BA_BLK


def _tc_body(cls_ref, code_ref, out_ref):
    j = pl.program_id(0)
    b = pl.program_id(1)
    p = cls_ref[0]
    p = jnp.minimum(jnp.maximum(p, 0.0001), 1.0 - 0.0001)
    code = code_ref[0]
    contrib = code > -0.5
    cols = lax.broadcasted_iota(jnp.int32, (BA_BLK, C), 1).astype(jnp.float32)
    is1 = code == cols
    u = jnp.where(is1, 1.0 - p, p)
    v = jnp.where(is1, p, 1.0 - p)
    af = jnp.where(is1, 0.95, 0.05)
    term = af * (u * u) * jnp.log(v)
    s = -jnp.sum(jnp.where(contrib, term, 0.0))

    @pl.when((b == 0) & (j == 0))
    def _init():
        out_ref[...] = jnp.zeros((8, 128), jnp.float32)

    rows = lax.broadcasted_iota(jnp.int32, (8, 128), 0)
    lanes = lax.broadcasted_iota(jnp.int32, (8, 128), 1)
    out_ref[...] = out_ref[...] + jnp.where((rows == j) & (lanes == 0), s, 0.0)


_focal_tc = pl.pallas_call(
    _tc_body,
    grid=(B, NBLK),
    in_specs=[pl.BlockSpec((1, BA_BLK, C), lambda j, b: (j, b, 0)),
              pl.BlockSpec((1, BA_BLK, 1), lambda j, b: (j, b, 0))],
    out_specs=pl.BlockSpec((8, 128), lambda j, b: (0, 0)),
    out_shape=jax.ShapeDtypeStruct((8, 128), jnp.float32),
)


def kernel(classifications, regressions, anchors, annotations, imgs, names):
    reg_flat = regressions.reshape(-1)
    anc_flat = anchors.reshape(-1)
    ann_flat = annotations.reshape(-1)
    code_flat, partials = _match_sc(reg_flat, anc_flat, ann_flat)
    cls_sums = _focal_tc(classifications, code_flat.reshape(B, A, 1))
    parts = partials.reshape(NW, B, 4).sum(axis=0)
    npos = parts[:, 1]
    cls = cls_sums[:B, 0] / jnp.maximum(npos, 1.0)
    xy = parts[:, 2] / jnp.maximum(2.0 * npos, 1.0)
    ang = parts[:, 3] / jnp.maximum(npos, 1.0)
    return (cls.mean(keepdims=True), xy.mean(keepdims=True),
            ang.mean(keepdims=True))


# speedup vs baseline: 1.0025x; 1.0025x over previous
import functools

import jax
import jax.numpy as jnp
from jax import lax
from jax.experimental import pallas as pl
from jax.experimental.pallas import tpu as pltpu
from jax.experimental.pallas import tpu_sc as plsc

B, A, C, M = 4, 50000, 16, 64
NW = 32
CHUNK = 1568
NSTRIP = CHUNK // 16
LAST_START = A - CHUNK


@functools.partial(
    pl.kernel,
    out_type=(jax.ShapeDtypeStruct((B * A,), jnp.float32),
              jax.ShapeDtypeStruct((NW * 16,), jnp.float32)),
    mesh=plsc.VectorSubcoreMesh(core_axis_name="c", subcore_axis_name="s"),
    scratch_types=[
        pltpu.VMEM((CHUNK * 3,), jnp.float32),
        pltpu.VMEM((CHUNK * 3,), jnp.float32),
        pltpu.VMEM((4 * M,), jnp.float32),
        pltpu.VMEM((CHUNK,), jnp.float32),
        pltpu.VMEM((16,), jnp.float32),
    ],
    compiler_params=pltpu.CompilerParams(needs_layout_passes=False),
)
def _match_sc(reg_hbm, anc_hbm, ann_hbm, code_hbm, out_hbm,
              anc_v, reg_v, ann_v, code_v, res_v):
    wid = lax.axis_index("s") * 2 + lax.axis_index("c")
    start = jnp.minimum(wid * CHUNK, LAST_START)
    own_lo = wid * CHUNK

    iota = lax.iota(jnp.int32, 16)
    zeros_i = iota * 0

    pltpu.sync_copy(anc_hbm.at[pl.ds(start * 3, CHUNK * 3)], anc_v)

    def strip_tail(base, aidx, d2min, bidx4, acc):
        npos_acc, xy_acc, ang_acc = acc
        aal = plsc.load_gather(anc_v, [aidx + 2])
        bx = plsc.load_gather(ann_v, [bidx4])
        by = plsc.load_gather(ann_v, [bidx4 + 1])
        bal = plsc.load_gather(ann_v, [bidx4 + 2])
        blb = plsc.load_gather(ann_v, [bidx4 + 3])
        aa = jnp.abs(aal - bal)

        pos_r = (d2min <= 25.0) & (aa <= 10.0)
        t0 = (d2min >= 56.25) | (aa >= 15.0)
        code = jnp.where(pos_r, blb, jnp.where(t0, 16.0, -1.0))
        code_v[pl.ds(base, 16)] = code

        validm = (start + base + iota) >= own_lo
        pos = pos_r & validm
        npos_acc = npos_acc + jnp.where(pos, 1.0, 0.0)

        ax = plsc.load_gather(anc_v, [aidx])
        ay = plsc.load_gather(anc_v, [aidx + 1])
        r0 = plsc.load_gather(reg_v, [aidx])
        r1 = plsc.load_gather(reg_v, [aidx + 1])
        r2 = plsc.load_gather(reg_v, [aidx + 2])
        dxr = jnp.abs((bx - ax) - r0)
        dyr = jnp.abs((by - ay) - r1)
        lx = jnp.where(dxr <= 1.0 / 9.0, 4.5 * dxr * dxr, dxr - 0.5 / 9.0)
        ly = jnp.where(dyr <= 1.0 / 9.0, 4.5 * dyr * dyr, dyr - 0.5 / 9.0)
        da = (jnp.abs((bal - aal) - r2) - 10.0) / 5.0
        da = jnp.where(da <= 0.0, 0.0, da)
        posf = jnp.where(pos, 1.0, 0.0)
        xy_acc = xy_acc + (lx + ly) * posf
        ang_acc = ang_acc + da * posf
        return npos_acc, xy_acc, ang_acc

    def batch_body(j, resvec):
        pltpu.sync_copy(reg_hbm.at[pl.ds(j * (3 * A) + start * 3, CHUNK * 3)], reg_v)
        pltpu.sync_copy(ann_hbm.at[pl.ds(j * (4 * M), 4 * M)], ann_v)

        def group_body(g, acc):
            base0 = g * 32
            base1 = base0 + 16
            aidx0 = iota * 3 + base0 * 3
            aidx1 = aidx0 + 48
            ax0 = plsc.load_gather(anc_v, [aidx0])
            ay0 = plsc.load_gather(anc_v, [aidx0 + 1])
            ax1 = plsc.load_gather(anc_v, [aidx1])
            ay1 = plsc.load_gather(anc_v, [aidx1 + 1])

            def m_body(m, mc):
                d0, b0, d1, b1 = mc
                mv = zeros_i + m * 4
                gx = plsc.load_gather(ann_v, [mv])
                gy = plsc.load_gather(ann_v, [mv + 1])
                dx0 = ax0 - gx
                dy0 = ay0 - gy
                dd0 = dx0 * dx0 + dy0 * dy0
                dx1 = ax1 - gx
                dy1 = ay1 - gy
                dd1 = dx1 * dx1 + dy1 * dy1
                lt0 = dd0 < d0
                lt1 = dd1 < d1
                return (jnp.where(lt0, dd0, d0), jnp.where(lt0, mv, b0),
                        jnp.where(lt1, dd1, d1), jnp.where(lt1, mv, b1))

            inf = jnp.full((16,), jnp.inf, jnp.float32)
            d0, b0, d1, b1 = lax.fori_loop(0, M, m_body,
                                           (inf, zeros_i, inf, zeros_i),
                                           unroll=8)
            acc = strip_tail(base0, aidx0, d0, b0, acc)
            acc = strip_tail(base1, aidx1, d1, b1, acc)
            return acc

        zf = jnp.zeros((16,), jnp.float32)
        npos_acc, xy_acc, ang_acc = lax.fori_loop(
            0, NSTRIP // 2, group_body, (zf, zf, zf))

        pltpu.sync_copy(code_v, code_hbm.at[pl.ds(j * A + start, CHUNK)])

        resvec = jnp.where(iota == 4 * j + 1, jnp.sum(npos_acc), resvec)
        resvec = jnp.where(iota == 4 * j + 2, jnp.sum(xy_acc), resvec)
        resvec = jnp.where(iota == 4 * j + 3, jnp.sum(ang_acc), resvec)
        return resvec

    res_v[...] = lax.fori_loop(0, B, batch_body, jnp.zeros((16,), jnp.float32))
    pltpu.sync_copy(res_v, out_hbm.at[pl.ds(wid * 16, 16)])


BA_BLK = 2000
NBLK = A // BA_BLK


def _tc_body(cls_ref, code_ref, out_ref):
    j = pl.program_id(0)
    b = pl.program_id(1)
    p = cls_ref[0]
    p = jnp.minimum(jnp.maximum(p, 0.0001), 1.0 - 0.0001)
    code = code_ref[0]
    contrib = code > -0.5
    cols = lax.broadcasted_iota(jnp.int32, (BA_BLK, C), 1).astype(jnp.float32)
    is1 = code == cols
    u = jnp.where(is1, 1.0 - p, p)
    v = jnp.where(is1, p, 1.0 - p)
    af = jnp.where(is1, 0.95, 0.05)
    term = af * (u * u) * jnp.log(v)
    s = -jnp.sum(jnp.where(contrib, term, 0.0))

    @pl.when((b == 0) & (j == 0))
    def _init():
        out_ref[...] = jnp.zeros((8, 128), jnp.float32)

    rows = lax.broadcasted_iota(jnp.int32, (8, 128), 0)
    lanes = lax.broadcasted_iota(jnp.int32, (8, 128), 1)
    out_ref[...] = out_ref[...] + jnp.where((rows == j) & (lanes == 0), s, 0.0)


_focal_tc = pl.pallas_call(
    _tc_body,
    grid=(B, NBLK),
    in_specs=[pl.BlockSpec((1, BA_BLK, C), lambda j, b: (j, b, 0)),
              pl.BlockSpec((1, BA_BLK, 1), lambda j, b: (j, b, 0))],
    out_specs=pl.BlockSpec((8, 128), lambda j, b: (0, 0)),
    out_shape=jax.ShapeDtypeStruct((8, 128), jnp.float32),
)


def kernel(classifications, regressions, anchors, annotations, imgs, names):
    reg_flat = regressions.reshape(-1)
    anc_flat = anchors.reshape(-1)
    ann_flat = annotations.reshape(-1)
    code_flat, partials = _match_sc(reg_flat, anc_flat, ann_flat)
    cls_sums = _focal_tc(classifications, code_flat.reshape(B, A, 1))
    parts = partials.reshape(NW, B, 4).sum(axis=0)
    npos = parts[:, 1]
    cls = cls_sums[:B, 0] / jnp.maximum(npos, 1.0)
    xy = parts[:, 2] / jnp.maximum(2.0 * npos, 1.0)
    ang = parts[:, 3] / jnp.maximum(npos, 1.0)
    return (cls.mean(keepdims=True), xy.mean(keepdims=True),
            ang.mean(keepdims=True))
